# baseline (speedup 1.0000x reference)
import functools

import jax
import jax.numpy as jnp
from jax import lax
from jax.experimental import pallas as pl
from jax.experimental.pallas import tpu as pltpu
from jax.experimental.pallas import tpu_sc as plsc

NUM_NUMERIC = 16
NUM_STYLES = 26
EMB = 64
BATCH = 16384
NFEAT = NUM_NUMERIC + NUM_STYLES

NC = 2
NS = 16
NW = NC * NS
B_PER_W = BATCH // NW
CH = 128
N_CH = B_PER_W // CH

BL = 512
HB = BL // 2


def _pack_body(n_rows, xt_ref, out_ref):
    x = xt_ref[...]
    ii = lax.broadcasted_iota(jnp.int32, (EMB, EMB), 0)
    jj = lax.broadcasted_iota(jnp.int32, (EMB, EMB), 1)
    eye = (ii == jj).astype(jnp.float32)
    y = lax.dot_general(x, eye, (((0,), (0,)), ((), ())),
                        preferred_element_type=jnp.float32)
    col = pl.program_id(0) * BL + lax.broadcasted_iota(jnp.int32, (BL, 1), 0)
    y = jnp.where(col < n_rows, y, 0.0)
    out_ref[...] = jnp.concatenate([y[:HB], y[HB:]], axis=1)


def _pack_table(table_t, n_rows):
    n_blk = (n_rows + BL - 1) // BL
    return pl.pallas_call(
        functools.partial(_pack_body, n_rows),
        grid=(n_blk,),
        in_specs=[pl.BlockSpec((EMB, BL), lambda i: (0, i))],
        out_specs=pl.BlockSpec((HB, 2 * EMB), lambda i: (i, 0)),
        out_shape=jax.ShapeDtypeStruct((n_blk * HB, 2 * EMB), jnp.float32),
        compiler_params=pltpu.CompilerParams(
            dimension_semantics=("parallel",),
        ),
    )(table_t)


def _sc_gather_body(uid_hbm, pid_hbm, ut_hbm, pt_hbm, uout_hbm, pout_hbm,
                    idx_u, idx_p, rows_u, rows_p, sem_u, sem_p):
    wid = lax.axis_index("s") * NC + lax.axis_index("c")
    base = wid * B_PER_W
    pltpu.sync_copy(uid_hbm.at[wid], idx_u)
    pltpu.sync_copy(pid_hbm.at[wid], idx_p)
    pend = [None, None]
    for j in range(N_CH + 2):
        b = j % 2
        if j >= 2:
            pu, pp = pend[b]
            pu.wait()
            pp.wait()
            pltpu.sync_copy(rows_u.at[b],
                            uout_hbm.at[pl.ds(base + (j - 2) * CH, CH)])
            pltpu.sync_copy(rows_p.at[b],
                            pout_hbm.at[pl.ds(base + (j - 2) * CH, CH)])
        if j < N_CH:
            pend[b] = (
                pltpu.async_copy(ut_hbm.at[idx_u.at[j]], rows_u.at[b], sem_u),
                pltpu.async_copy(pt_hbm.at[idx_p.at[j]], rows_p.at[b], sem_p),
            )


@functools.cache
def _sc_gather():
    mesh = plsc.VectorSubcoreMesh(core_axis_name="c", subcore_axis_name="s")
    return pl.kernel(
        _sc_gather_body,
        out_type=[
            jax.ShapeDtypeStruct((BATCH, 2 * EMB), jnp.float32),
            jax.ShapeDtypeStruct((BATCH, 2 * EMB), jnp.float32),
        ],
        mesh=mesh,
        scratch_types=[
            pltpu.VMEM((N_CH, CH), jnp.int32),
            pltpu.VMEM((N_CH, CH), jnp.int32),
            pltpu.VMEM((2, CH, 2 * EMB), jnp.float32),
            pltpu.VMEM((2, CH, 2 * EMB), jnp.float32),
            pltpu.SemaphoreType.DMA,
            pltpu.SemaphoreType.DMA,
        ],
        compiler_params=pltpu.CompilerParams(use_tc_tiling_on_sc=False),
    )


TB = 2048


def _dot_t(lhs_t, rhs):
    return lax.dot_general(lhs_t, rhs, (((0,), (0,)), ((), ())),
                           preferred_element_type=jnp.float32)


def _dot_nt(lhs, rhs_t):
    return lax.dot_general(lhs, rhs_t, (((1,), (1,)), ((), ())),
                           preferred_element_type=jnp.float32)


def _mlp_body(uv_ref, pv_ref, hu_ref, hp_ref, fft_ref, wfeat_ref, bns_ref,
              w1u2_ref, w1p2_ref, w1ns_ref, b1_ref,
              w2t_ref, b2_ref, w3t_ref, b3_ref, wf_ref, bf_ref, out_ref):
    lane_half = lax.broadcasted_iota(jnp.int32, (1, 2 * EMB), 1) // EMB
    mask_u = (lane_half == hu_ref[...]).astype(jnp.float32)
    mask_p = (lane_half == hp_ref[...]).astype(jnp.float32)
    ns = jnp.maximum(_dot_t(fft_ref[...], wfeat_ref[...]) + bns_ref[...], 0.0)
    h = (uv_ref[...] * mask_u) @ w1u2_ref[...]
    h = h + (pv_ref[...] * mask_p) @ w1p2_ref[...]
    h = h + ns @ w1ns_ref[...]
    h1 = jnp.maximum(h + b1_ref[...], 0.0)
    h2 = jnp.maximum(_dot_nt(h1, w2t_ref[...]) + b2_ref[...], 0.0)
    h3 = jnp.maximum(_dot_nt(h2, w3t_ref[...]) + b3_ref[...], 0.0)
    z = jnp.sum(h3 * wf_ref[...], axis=1, keepdims=True) + bf_ref[0, 0]
    out_ref[...] = 1.0 / (1.0 + jnp.exp(-z))


def _mlp(uv, pv, hu, hp, fft, wfeat, bns, w1u2, w1p2, w1ns, b1,
         w2t, b2, w3t, b3, wf, bf):
    grid = (BATCH // TB,)
    full = lambda shape: pl.BlockSpec(shape, lambda i: (0, 0))
    return pl.pallas_call(
        _mlp_body,
        grid=grid,
        in_specs=[
            pl.BlockSpec((TB, 2 * EMB), lambda i: (i, 0)),
            pl.BlockSpec((TB, 2 * EMB), lambda i: (i, 0)),
            pl.BlockSpec((TB, 1), lambda i: (i, 0)),
            pl.BlockSpec((TB, 1), lambda i: (i, 0)),
            pl.BlockSpec((NFEAT, TB), lambda i: (0, i)),
            full((NFEAT, 2 * EMB)),
            full((1, 2 * EMB)),
            full((2 * EMB, 128)),
            full((2 * EMB, 128)),
            full((2 * EMB, 128)),
            full((1, 128)),
            full((64, 128)),
            full((1, 64)),
            full((32, 64)),
            full((1, 32)),
            full((1, 32)),
            full((1, 1)),
        ],
        out_specs=pl.BlockSpec((TB, 1), lambda i: (i, 0)),
        out_shape=jax.ShapeDtypeStruct((BATCH, 1), jnp.float32),
        compiler_params=pltpu.CompilerParams(
            dimension_semantics=("parallel",),
        ),
    )(uv, pv, hu, hp, fft, wfeat, bns, w1u2, w1p2, w1ns, b1,
      w2t, b2, w3t, b3, wf, bf)


def kernel(user_id, product_id, full_features, user_table, product_table,
           W_num, b_num, W_style, b_style, W1, b1, W2, b2, W3, b3, Wf, bf):
    uid = user_id.astype(jnp.int32)
    pid = product_id.astype(jnp.int32)
    pidx_u = (uid // BL) * HB + uid % HB
    pidx_p = (pid // BL) * HB + pid % HB
    half_u = ((uid // HB) % 2).reshape(BATCH, 1)
    half_p = ((pid // HB) % 2).reshape(BATCH, 1)

    ut_pk = _pack_table(user_table.T, user_table.shape[0])
    pt_pk = _pack_table(product_table.T, product_table.shape[0])
    uvec, pvec = _sc_gather()(pidx_u.reshape(NW, N_CH, CH),
                              pidx_p.reshape(NW, N_CH, CH), ut_pk, pt_pk)

    wfeat = jnp.zeros((NFEAT, 2 * EMB), jnp.float32)
    wfeat = wfeat.at[:NUM_NUMERIC, :EMB].set(W_num)
    wfeat = wfeat.at[NUM_NUMERIC:, EMB:].set(W_style)
    bns = jnp.concatenate([b_num, b_style])[None, :]
    w1u2 = jnp.concatenate([W1[:EMB], W1[:EMB]], axis=0)
    w1p2 = jnp.concatenate([W1[EMB:2 * EMB], W1[EMB:2 * EMB]], axis=0)

    return _mlp(uvec, pvec, half_u, half_p, full_features.T, wfeat, bns,
                w1u2, w1p2, W1[2 * EMB:], b1[None, :],
                W2.T, b2[None, :], W3.T, b3[None, :],
                Wf.reshape(1, 32), bf.reshape(1, 1))


# speedup vs baseline: 4.6109x; 4.6109x over previous
import functools

import jax
import jax.numpy as jnp
from jax import lax
from jax.experimental import pallas as pl
from jax.experimental.pallas import tpu as pltpu
from jax.experimental.pallas import tpu_sc as plsc

NUM_NUMERIC = 16
NUM_STYLES = 26
EMB = 64
BATCH = 16384
NFEAT = NUM_NUMERIC + NUM_STYLES

NC = 2
NS = 16
NW = NC * NS
B_PER_W = BATCH // NW
CH = 128
N_CH = B_PER_W // CH

BL = 32768
HB = BL // 2


def _pack_body(n_rows, xt_ref, out_ref):
    x = xt_ref[...]
    ii = lax.broadcasted_iota(jnp.int32, (EMB, EMB), 0)
    jj = lax.broadcasted_iota(jnp.int32, (EMB, EMB), 1)
    eye = (ii == jj).astype(jnp.float32)
    y = lax.dot_general(x, eye, (((0,), (0,)), ((), ())),
                        preferred_element_type=jnp.float32)
    col = pl.program_id(0) * BL + lax.broadcasted_iota(jnp.int32, (BL, 1), 0)
    y = jnp.where(col < n_rows, y, 0.0)
    out_ref[...] = jnp.concatenate([y[:HB], y[HB:]], axis=1)


def _pack_table(table_t, n_rows):
    n_blk = (n_rows + BL - 1) // BL
    return pl.pallas_call(
        functools.partial(_pack_body, n_rows),
        grid=(n_blk,),
        in_specs=[pl.BlockSpec((EMB, BL), lambda i: (0, i))],
        out_specs=pl.BlockSpec((HB, 2 * EMB), lambda i: (i, 0)),
        out_shape=jax.ShapeDtypeStruct((n_blk * HB, 2 * EMB), jnp.float32),
        compiler_params=pltpu.CompilerParams(
            dimension_semantics=("parallel",),
        ),
    )(table_t)


def _sc_gather_body(uid_hbm, pid_hbm, ut_hbm, pt_hbm, uout_hbm, pout_hbm,
                    idx_u, idx_p, rows_u, rows_p, sem_u, sem_p):
    wid = lax.axis_index("s") * NC + lax.axis_index("c")
    base = wid * B_PER_W
    pltpu.sync_copy(uid_hbm.at[wid], idx_u)
    pltpu.sync_copy(pid_hbm.at[wid], idx_p)
    pend = [None, None]
    for j in range(N_CH + 2):
        b = j % 2
        if j >= 2:
            pu, pp = pend[b]
            pu.wait()
            pp.wait()
            pltpu.sync_copy(rows_u.at[b],
                            uout_hbm.at[pl.ds(base + (j - 2) * CH, CH)])
            pltpu.sync_copy(rows_p.at[b],
                            pout_hbm.at[pl.ds(base + (j - 2) * CH, CH)])
        if j < N_CH:
            pend[b] = (
                pltpu.async_copy(ut_hbm.at[idx_u.at[j]], rows_u.at[b], sem_u),
                pltpu.async_copy(pt_hbm.at[idx_p.at[j]], rows_p.at[b], sem_p),
            )


@functools.cache
def _sc_gather():
    mesh = plsc.VectorSubcoreMesh(core_axis_name="c", subcore_axis_name="s")
    return pl.kernel(
        _sc_gather_body,
        out_type=[
            jax.ShapeDtypeStruct((BATCH, 2 * EMB), jnp.float32),
            jax.ShapeDtypeStruct((BATCH, 2 * EMB), jnp.float32),
        ],
        mesh=mesh,
        scratch_types=[
            pltpu.VMEM((N_CH, CH), jnp.int32),
            pltpu.VMEM((N_CH, CH), jnp.int32),
            pltpu.VMEM((2, CH, 2 * EMB), jnp.float32),
            pltpu.VMEM((2, CH, 2 * EMB), jnp.float32),
            pltpu.SemaphoreType.DMA,
            pltpu.SemaphoreType.DMA,
        ],
        compiler_params=pltpu.CompilerParams(use_tc_tiling_on_sc=False),
    )


TB = 2048


def _dot_t(lhs_t, rhs):
    return lax.dot_general(lhs_t, rhs, (((0,), (0,)), ((), ())),
                           preferred_element_type=jnp.float32)


def _dot_nt(lhs, rhs_t):
    return lax.dot_general(lhs, rhs_t, (((1,), (1,)), ((), ())),
                           preferred_element_type=jnp.float32)


def _mlp_body(uv_ref, pv_ref, hu_ref, hp_ref, fft_ref, wfeat_ref, bns_ref,
              w1u2_ref, w1p2_ref, w1ns_ref, b1_ref,
              w2t_ref, b2_ref, w3t_ref, b3_ref, wf_ref, bf_ref, out_ref):
    lane_half = lax.broadcasted_iota(jnp.int32, (1, 2 * EMB), 1) // EMB
    mask_u = (lane_half == hu_ref[...]).astype(jnp.float32)
    mask_p = (lane_half == hp_ref[...]).astype(jnp.float32)
    ns = jnp.maximum(_dot_t(fft_ref[...], wfeat_ref[...]) + bns_ref[...], 0.0)
    h = (uv_ref[...] * mask_u) @ w1u2_ref[...]
    h = h + (pv_ref[...] * mask_p) @ w1p2_ref[...]
    h = h + ns @ w1ns_ref[...]
    h1 = jnp.maximum(h + b1_ref[...], 0.0)
    h2 = jnp.maximum(_dot_nt(h1, w2t_ref[...]) + b2_ref[...], 0.0)
    h3 = jnp.maximum(_dot_nt(h2, w3t_ref[...]) + b3_ref[...], 0.0)
    z = jnp.sum(h3 * wf_ref[...], axis=1, keepdims=True) + bf_ref[0, 0]
    out_ref[...] = 1.0 / (1.0 + jnp.exp(-z))


def _mlp(uv, pv, hu, hp, fft, wfeat, bns, w1u2, w1p2, w1ns, b1,
         w2t, b2, w3t, b3, wf, bf):
    grid = (BATCH // TB,)
    full = lambda shape: pl.BlockSpec(shape, lambda i: (0, 0))
    return pl.pallas_call(
        _mlp_body,
        grid=grid,
        in_specs=[
            pl.BlockSpec((TB, 2 * EMB), lambda i: (i, 0)),
            pl.BlockSpec((TB, 2 * EMB), lambda i: (i, 0)),
            pl.BlockSpec((TB, 1), lambda i: (i, 0)),
            pl.BlockSpec((TB, 1), lambda i: (i, 0)),
            pl.BlockSpec((NFEAT, TB), lambda i: (0, i)),
            full((NFEAT, 2 * EMB)),
            full((1, 2 * EMB)),
            full((2 * EMB, 128)),
            full((2 * EMB, 128)),
            full((2 * EMB, 128)),
            full((1, 128)),
            full((64, 128)),
            full((1, 64)),
            full((32, 64)),
            full((1, 32)),
            full((1, 32)),
            full((1, 1)),
        ],
        out_specs=pl.BlockSpec((TB, 1), lambda i: (i, 0)),
        out_shape=jax.ShapeDtypeStruct((BATCH, 1), jnp.float32),
        compiler_params=pltpu.CompilerParams(
            dimension_semantics=("parallel",),
        ),
    )(uv, pv, hu, hp, fft, wfeat, bns, w1u2, w1p2, w1ns, b1,
      w2t, b2, w3t, b3, wf, bf)


def kernel(user_id, product_id, full_features, user_table, product_table,
           W_num, b_num, W_style, b_style, W1, b1, W2, b2, W3, b3, Wf, bf):
    uid = user_id.astype(jnp.int32)
    pid = product_id.astype(jnp.int32)
    pidx_u = (uid // BL) * HB + uid % HB
    pidx_p = (pid // BL) * HB + pid % HB
    half_u = ((uid // HB) % 2).reshape(BATCH, 1)
    half_p = ((pid // HB) % 2).reshape(BATCH, 1)

    ut_pk = _pack_table(user_table.T, user_table.shape[0])
    pt_pk = _pack_table(product_table.T, product_table.shape[0])
    uvec, pvec = _sc_gather()(pidx_u.reshape(NW, N_CH, CH),
                              pidx_p.reshape(NW, N_CH, CH), ut_pk, pt_pk)

    wfeat = jnp.zeros((NFEAT, 2 * EMB), jnp.float32)
    wfeat = wfeat.at[:NUM_NUMERIC, :EMB].set(W_num)
    wfeat = wfeat.at[NUM_NUMERIC:, EMB:].set(W_style)
    bns = jnp.concatenate([b_num, b_style])[None, :]
    w1u2 = jnp.concatenate([W1[:EMB], W1[:EMB]], axis=0)
    w1p2 = jnp.concatenate([W1[EMB:2 * EMB], W1[EMB:2 * EMB]], axis=0)

    return _mlp(uvec, pvec, half_u, half_p, full_features.T, wfeat, bns,
                w1u2, w1p2, W1[2 * EMB:], b1[None, :],
                W2.T, b2[None, :], W3.T, b3[None, :],
                Wf.reshape(1, 32), bf.reshape(1, 1))


# speedup vs baseline: 5.9614x; 1.2929x over previous
import functools

import jax
import jax.numpy as jnp
from jax import lax
from jax.experimental import pallas as pl
from jax.experimental.pallas import tpu as pltpu
from jax.experimental.pallas import tpu_sc as plsc

NUM_NUMERIC = 16
NUM_STYLES = 26
EMB = 64
BATCH = 16384
NFEAT = NUM_NUMERIC + NUM_STYLES

NC = 2
NS = 16
NW = NC * NS
B_PER_W = BATCH // NW
CH = 128
N_CH = B_PER_W // CH

BL = 32768
HB = BL // 2


def _pack_body(n_rows, xt_ref, out_ref):
    x = xt_ref[...]
    lane = lax.broadcasted_iota(jnp.int32, (1, HB), 1) + pl.program_id(0) * BL
    xa = jnp.where(lane < n_rows, x[:, :HB], 0.0)
    xb = jnp.where(lane + HB < n_rows, x[:, HB:], 0.0)
    xs = jnp.concatenate([xa, xb], axis=0)
    ii = lax.broadcasted_iota(jnp.int32, (2 * EMB, 2 * EMB), 0)
    jj = lax.broadcasted_iota(jnp.int32, (2 * EMB, 2 * EMB), 1)
    eye = (ii == jj).astype(jnp.float32)
    out_ref[...] = lax.dot_general(xs, eye, (((0,), (0,)), ((), ())),
                                   preferred_element_type=jnp.float32)


def _pack_table(table_t, n_rows):
    n_blk = (n_rows + BL - 1) // BL
    return pl.pallas_call(
        functools.partial(_pack_body, n_rows),
        grid=(n_blk,),
        in_specs=[pl.BlockSpec((EMB, BL), lambda i: (0, i))],
        out_specs=pl.BlockSpec((HB, 2 * EMB), lambda i: (i, 0)),
        out_shape=jax.ShapeDtypeStruct((n_blk * HB, 2 * EMB), jnp.float32),
        compiler_params=pltpu.CompilerParams(
            dimension_semantics=("parallel",),
        ),
    )(table_t)


def _sc_gather_body(uid_hbm, pid_hbm, ut_hbm, pt_hbm, uout_hbm, pout_hbm,
                    idx_u, idx_p, rows_u, rows_p, sem_u, sem_p):
    wid = lax.axis_index("s") * NC + lax.axis_index("c")
    base = wid * B_PER_W
    pltpu.sync_copy(uid_hbm.at[wid], idx_u)
    pltpu.sync_copy(pid_hbm.at[wid], idx_p)
    pend = [None, None]
    for j in range(N_CH + 2):
        b = j % 2
        if j >= 2:
            pu, pp = pend[b]
            pu.wait()
            pp.wait()
            pltpu.sync_copy(rows_u.at[b],
                            uout_hbm.at[pl.ds(base + (j - 2) * CH, CH)])
            pltpu.sync_copy(rows_p.at[b],
                            pout_hbm.at[pl.ds(base + (j - 2) * CH, CH)])
        if j < N_CH:
            pend[b] = (
                pltpu.async_copy(ut_hbm.at[idx_u.at[j]], rows_u.at[b], sem_u),
                pltpu.async_copy(pt_hbm.at[idx_p.at[j]], rows_p.at[b], sem_p),
            )


@functools.cache
def _sc_gather():
    mesh = plsc.VectorSubcoreMesh(core_axis_name="c", subcore_axis_name="s")
    return pl.kernel(
        _sc_gather_body,
        out_type=[
            jax.ShapeDtypeStruct((BATCH, 2 * EMB), jnp.float32),
            jax.ShapeDtypeStruct((BATCH, 2 * EMB), jnp.float32),
        ],
        mesh=mesh,
        scratch_types=[
            pltpu.VMEM((N_CH, CH), jnp.int32),
            pltpu.VMEM((N_CH, CH), jnp.int32),
            pltpu.VMEM((2, CH, 2 * EMB), jnp.float32),
            pltpu.VMEM((2, CH, 2 * EMB), jnp.float32),
            pltpu.SemaphoreType.DMA,
            pltpu.SemaphoreType.DMA,
        ],
        compiler_params=pltpu.CompilerParams(use_tc_tiling_on_sc=False),
    )


TB = 2048


def _dot_t(lhs_t, rhs):
    return lax.dot_general(lhs_t, rhs, (((0,), (0,)), ((), ())),
                           preferred_element_type=jnp.float32)


def _dot_nt(lhs, rhs_t):
    return lax.dot_general(lhs, rhs_t, (((1,), (1,)), ((), ())),
                           preferred_element_type=jnp.float32)


def _mlp_body(uv_ref, pv_ref, hu_ref, hp_ref, fft_ref, wfeat_ref, bns_ref,
              w1u2_ref, w1p2_ref, w1ns_ref, b1_ref,
              w2t_ref, b2_ref, w3t_ref, b3_ref, wf_ref, bf_ref, out_ref):
    lane_half = lax.broadcasted_iota(jnp.int32, (1, 2 * EMB), 1) // EMB
    mask_u = (lane_half == hu_ref[...]).astype(jnp.float32)
    mask_p = (lane_half == hp_ref[...]).astype(jnp.float32)
    ns = jnp.maximum(_dot_t(fft_ref[...], wfeat_ref[...]) + bns_ref[...], 0.0)
    h = (uv_ref[...] * mask_u) @ w1u2_ref[...]
    h = h + (pv_ref[...] * mask_p) @ w1p2_ref[...]
    h = h + ns @ w1ns_ref[...]
    h1 = jnp.maximum(h + b1_ref[...], 0.0)
    h2 = jnp.maximum(_dot_nt(h1, w2t_ref[...]) + b2_ref[...], 0.0)
    h3 = jnp.maximum(_dot_nt(h2, w3t_ref[...]) + b3_ref[...], 0.0)
    z = jnp.sum(h3 * wf_ref[...], axis=1, keepdims=True) + bf_ref[0, 0]
    out_ref[...] = 1.0 / (1.0 + jnp.exp(-z))


def _mlp(uv, pv, hu, hp, fft, wfeat, bns, w1u2, w1p2, w1ns, b1,
         w2t, b2, w3t, b3, wf, bf):
    grid = (BATCH // TB,)
    full = lambda shape: pl.BlockSpec(shape, lambda i: (0, 0))
    return pl.pallas_call(
        _mlp_body,
        grid=grid,
        in_specs=[
            pl.BlockSpec((TB, 2 * EMB), lambda i: (i, 0)),
            pl.BlockSpec((TB, 2 * EMB), lambda i: (i, 0)),
            pl.BlockSpec((TB, 1), lambda i: (i, 0)),
            pl.BlockSpec((TB, 1), lambda i: (i, 0)),
            pl.BlockSpec((NFEAT, TB), lambda i: (0, i)),
            full((NFEAT, 2 * EMB)),
            full((1, 2 * EMB)),
            full((2 * EMB, 128)),
            full((2 * EMB, 128)),
            full((2 * EMB, 128)),
            full((1, 128)),
            full((64, 128)),
            full((1, 64)),
            full((32, 64)),
            full((1, 32)),
            full((1, 32)),
            full((1, 1)),
        ],
        out_specs=pl.BlockSpec((TB, 1), lambda i: (i, 0)),
        out_shape=jax.ShapeDtypeStruct((BATCH, 1), jnp.float32),
        compiler_params=pltpu.CompilerParams(
            dimension_semantics=("parallel",),
        ),
    )(uv, pv, hu, hp, fft, wfeat, bns, w1u2, w1p2, w1ns, b1,
      w2t, b2, w3t, b3, wf, bf)


def kernel(user_id, product_id, full_features, user_table, product_table,
           W_num, b_num, W_style, b_style, W1, b1, W2, b2, W3, b3, Wf, bf):
    uid = user_id.astype(jnp.int32)
    pid = product_id.astype(jnp.int32)
    pidx_u = (uid // BL) * HB + uid % HB
    pidx_p = (pid // BL) * HB + pid % HB
    half_u = ((uid // HB) % 2).reshape(BATCH, 1)
    half_p = ((pid // HB) % 2).reshape(BATCH, 1)

    ut_pk = _pack_table(user_table.T, user_table.shape[0])
    pt_pk = _pack_table(product_table.T, product_table.shape[0])
    uvec, pvec = _sc_gather()(pidx_u.reshape(NW, N_CH, CH),
                              pidx_p.reshape(NW, N_CH, CH), ut_pk, pt_pk)

    wfeat = jnp.zeros((NFEAT, 2 * EMB), jnp.float32)
    wfeat = wfeat.at[:NUM_NUMERIC, :EMB].set(W_num)
    wfeat = wfeat.at[NUM_NUMERIC:, EMB:].set(W_style)
    bns = jnp.concatenate([b_num, b_style])[None, :]
    w1u2 = jnp.concatenate([W1[:EMB], W1[:EMB]], axis=0)
    w1p2 = jnp.concatenate([W1[EMB:2 * EMB], W1[EMB:2 * EMB]], axis=0)

    return _mlp(uvec, pvec, half_u, half_p, full_features.T, wfeat, bns,
                w1u2, w1p2, W1[2 * EMB:], b1[None, :],
                W2.T, b2[None, :], W3.T, b3[None, :],
                Wf.reshape(1, 32), bf.reshape(1, 1))
